# baseline (speedup 1.0000x reference)
import jax
import jax.numpy as jnp
from jax import lax
from jax.experimental import pallas as pl
from jax.experimental.pallas import tpu as pltpu
from jax.experimental.pallas import tpu_sc as plsc

N_ROWS = 10000
D = 128
E = 320000

NUM_CORES = 2
NUM_SUBCORES = 16
CHUNK = 64
NCH = 316
E_PAD = NUM_SUBCORES * CHUNK * NCH
ROWS_PER_TILE = N_ROWS // NUM_SUBCORES
COL, ROW, VAL = 0, 1, 2


def _spmm_body(emb_hbm, recs_hbm, zeros_hbm, out_hbm,
               ibuf0, ibuf1, ibuf2, ibuf3, gbuf0, gbuf1, gbuf2, gbuf3,
               sbuf0, sbuf1, accum,
               isem0, isem1, isem2, isem3, gsem0, gsem1, gsem2, gsem3,
               ssem0, ssem1):
    c = lax.axis_index("c")
    s = lax.axis_index("s")
    ibufs = (ibuf0, ibuf1, ibuf2, ibuf3)
    gbufs = (gbuf0, gbuf1, gbuf2, gbuf3)
    sbufs = (sbuf0, sbuf1)
    isems = (isem0, isem1, isem2, isem3)
    gsems = (gsem0, gsem1, gsem2, gsem3)
    ssems = (ssem0, ssem1)

    pltpu.sync_copy(zeros_hbm, accum.at[pl.ds(s * ROWS_PER_TILE, ROWS_PER_TILE)])

    pltpu.async_copy(recs_hbm.at[c, s, 0], ibuf0, isem0)
    pltpu.async_copy(recs_hbm.at[c, s, 1], ibuf1, isem1)
    plsc.subcore_barrier()
    pltpu.make_async_copy(recs_hbm.at[c, s, 0], ibuf0, isem0).wait()
    pltpu.async_copy(emb_hbm.at[ibuf0.at[COL]], gbuf0, gsem0)

    def quad_body(mm, carry):
        for k in range(4):
            m = mm * 4 + k
            k1 = (k + 1) % 4
            k2 = (k + 2) % 4
            ks = k % 2

            @pl.when(mm * 4 + k + 1 < NCH)
            def _(_k1=k1):
                pltpu.make_async_copy(
                    recs_hbm.at[c, s, 0], ibufs[_k1], isems[_k1]).wait()
                pltpu.async_copy(
                    emb_hbm.at[ibufs[_k1].at[COL]], gbufs[_k1], gsems[_k1])

            pltpu.make_async_copy(
                emb_hbm.at[ibufs[k].at[COL]], gbufs[k], gsems[k]).wait()

            @pl.when(mm * 4 + k >= 2)
            def _(_k=k, _ks=ks):
                pltpu.make_async_copy(
                    sbufs[_ks], accum.at[ibufs[_k].at[ROW]], ssems[_ks]).wait()

            @plsc.parallel_loop(0, CHUNK // 16, unroll=1)
            def group_body(g, _k=k, _ks=ks):
                vvec = lax.bitcast_convert_type(
                    ibufs[_k][VAL, pl.ds(g * 16, 16)], jnp.float32)
                for lane in range(16):
                    vv = jnp.full((16,), vvec[lane], dtype=jnp.float32)
                    e = g * 16 + lane
                    for d in range(D // 16):
                        sl = pl.ds(d * 16, 16)
                        sbufs[_ks][e, sl] = gbufs[_k][e, sl] * vv

            pltpu.async_copy(sbufs[ks], accum.at[ibufs[k].at[ROW]],
                             ssems[ks], add=True)

            @pl.when(mm * 4 + k + 2 < NCH)
            def _(_k2=k2, _m=m):
                pltpu.async_copy(recs_hbm.at[c, s, _m + 2], ibufs[_k2], isems[_k2])
        return carry

    lax.fori_loop(0, NCH // 4, quad_body, 0, unroll=False)

    for ks in range(2):
        pltpu.make_async_copy(
            sbufs[ks], accum.at[ibufs[0].at[ROW]], ssems[ks]).wait()

    plsc.subcore_barrier()

    pltpu.sync_copy(
        accum.at[pl.ds(s * ROWS_PER_TILE, ROWS_PER_TILE)],
        out_hbm.at[c, s],
    )


@jax.jit
def kernel(users_emb, items_emb, user_indices, user_values, item_indices, item_values):
    emb = jnp.concatenate([users_emb, items_emb], axis=0)

    def prep(a):
        a = a.astype(jnp.int32)
        a = jnp.concatenate([a, jnp.zeros((E_PAD - E,), jnp.int32)])
        return a.reshape(NUM_SUBCORES, NCH, 1, CHUNK)

    recs = jnp.stack([
        jnp.concatenate([
            prep(user_indices[1]),
            prep(user_indices[0]),
            prep(lax.bitcast_convert_type(user_values, jnp.int32)),
        ], axis=2),
        jnp.concatenate([
            prep(item_indices[1] + N_ROWS),
            prep(item_indices[0]),
            prep(lax.bitcast_convert_type(item_values, jnp.int32)),
        ], axis=2),
    ])
    zeros = jnp.zeros((ROWS_PER_TILE, D), jnp.float32)

    mesh = plsc.VectorSubcoreMesh(
        core_axis_name="c", subcore_axis_name="s",
        num_cores=NUM_CORES, num_subcores=NUM_SUBCORES,
    )
    out = pl.kernel(
        _spmm_body,
        out_type=jax.ShapeDtypeStruct(
            (NUM_CORES, NUM_SUBCORES, ROWS_PER_TILE, D), jnp.float32),
        mesh=mesh,
        compiler_params=pltpu.CompilerParams(use_tc_tiling_on_sc=False),
        scratch_types=(
            [pltpu.VMEM((3, CHUNK), jnp.int32) for _ in range(4)]
            + [pltpu.VMEM((CHUNK, D), jnp.float32) for _ in range(4)]
            + [pltpu.VMEM((CHUNK, D), jnp.float32) for _ in range(2)]
            + [pltpu.VMEM_SHARED((N_ROWS, D), jnp.float32)]
            + [pltpu.SemaphoreType.DMA] * 10
        ),
    )(emb, recs, zeros)

    out = out.reshape(NUM_CORES, N_ROWS, D)
    return (out[0], out[1])


# speedup vs baseline: 1.0608x; 1.0608x over previous
import jax
import jax.numpy as jnp
from jax import lax
from jax.experimental import pallas as pl
from jax.experimental.pallas import tpu as pltpu
from jax.experimental.pallas import tpu_sc as plsc

N_ROWS = 10000
D = 128
E = 320000

NUM_CORES = 2
NUM_SUBCORES = 16
CHUNK = 128
NCH = 158
E_PAD = NUM_SUBCORES * CHUNK * NCH
ROWS_PER_TILE = N_ROWS // NUM_SUBCORES
COL, ROW, VAL = 0, 1, 2
WPR = D // 2


def _spmm_body(emb_hbm, recs_hbm, zeros_hbm, out_hbm,
               ibuf0, ibuf1, gbuf0, gbuf1, sbuf0, sbuf1, scatidx, accum,
               isem0, isem1, gsem0, gsem1, ssem0, ssem1):
    c = lax.axis_index("c")
    s = lax.axis_index("s")
    ibufs = (ibuf0, ibuf1)
    gbufs = (gbuf0, gbuf1)
    sbufs = (sbuf0, sbuf1)
    isems = (isem0, isem1)
    gsems = (gsem0, gsem1)
    ssems = (ssem0, ssem1)
    himask = jnp.full((16,), -65536, dtype=jnp.int32)

    pltpu.sync_copy(zeros_hbm, accum.at[pl.ds(s * ROWS_PER_TILE, ROWS_PER_TILE)])

    pltpu.async_copy(recs_hbm.at[c, s, 0], ibuf0, isem0)
    pltpu.async_copy(recs_hbm.at[c, s, 1], ibuf1, isem1)
    plsc.subcore_barrier()
    pltpu.make_async_copy(recs_hbm.at[c, s, 0], ibuf0, isem0).wait()
    pltpu.async_copy(emb_hbm.at[ibuf0.at[COL]], gbuf0, gsem0)

    def pair_body(jj, carry):
        for b in range(2):
            m = jj * 2 + b
            o = 1 - b

            @pl.when(jj * 2 + b + 1 < NCH)
            def _(_o=o):
                pltpu.make_async_copy(
                    recs_hbm.at[c, s, 0], ibufs[_o], isems[_o]).wait()
                pltpu.async_copy(
                    emb_hbm.at[ibufs[_o].at[COL]], gbufs[_o], gsems[_o])

            pltpu.make_async_copy(
                emb_hbm.at[ibufs[b].at[COL]], gbufs[b], gsems[b]).wait()

            @pl.when(jj * 2 + b >= 2)
            def _(_b=b):
                pltpu.make_async_copy(
                    sbufs[_b], accum.at[scatidx.at[_b]], ssems[_b]).wait()

            @plsc.parallel_loop(0, CHUNK // 16, unroll=1)
            def group_body(g, _b=b):
                vvec = lax.bitcast_convert_type(
                    ibufs[_b][VAL, pl.ds(g * 16, 16)], jnp.float32)
                for lane in range(16):
                    vv = jnp.full((16,), vvec[lane], dtype=jnp.float32)
                    e = g * 16 + lane
                    for d in range(WPR // 16):
                        w = gbufs[_b][e, pl.ds(d * 16, 16)]
                        lo = lax.bitcast_convert_type(
                            lax.shift_left(w, 16), jnp.float32)
                        hi = lax.bitcast_convert_type(w & himask, jnp.float32)
                        sbufs[_b][e, pl.ds(d * 32, 16)] = lo * vv
                        sbufs[_b][e, pl.ds(d * 32 + 16, 16)] = hi * vv

            for q in range(CHUNK // 16):
                scatidx[b, pl.ds(q * 16, 16)] = ibufs[b][ROW, pl.ds(q * 16, 16)]
            pltpu.async_copy(sbufs[b], accum.at[scatidx.at[b]], ssems[b],
                             add=True)

            @pl.when(jj * 2 + b + 2 < NCH)
            def _(_b=b, _m=m):
                pltpu.async_copy(recs_hbm.at[c, s, _m + 2], ibufs[_b], isems[_b])
        return carry

    lax.fori_loop(0, NCH // 2, pair_body, 0, unroll=False)

    for b in range(2):
        pltpu.make_async_copy(
            sbufs[b], accum.at[scatidx.at[b]], ssems[b]).wait()

    plsc.subcore_barrier()

    pltpu.sync_copy(
        accum.at[pl.ds(s * ROWS_PER_TILE, ROWS_PER_TILE)],
        out_hbm.at[c, s],
    )


@jax.jit
def kernel(users_emb, items_emb, user_indices, user_values, item_indices, item_values):
    emb = jnp.concatenate([users_emb, items_emb], axis=0)
    emb = lax.bitcast_convert_type(
        emb.astype(jnp.bfloat16).reshape(2 * N_ROWS, WPR, 2), jnp.int32)

    def prep(a):
        a = a.astype(jnp.int32)
        a = jnp.concatenate([a, jnp.zeros((E_PAD - E,), jnp.int32)])
        return a.reshape(NUM_SUBCORES, NCH, 1, CHUNK)

    recs = jnp.stack([
        jnp.concatenate([
            prep(user_indices[1]),
            prep(user_indices[0]),
            prep(lax.bitcast_convert_type(user_values, jnp.int32)),
        ], axis=2),
        jnp.concatenate([
            prep(item_indices[1] + N_ROWS),
            prep(item_indices[0]),
            prep(lax.bitcast_convert_type(item_values, jnp.int32)),
        ], axis=2),
    ])
    zeros = jnp.zeros((ROWS_PER_TILE, D), jnp.float32)

    mesh = plsc.VectorSubcoreMesh(
        core_axis_name="c", subcore_axis_name="s",
        num_cores=NUM_CORES, num_subcores=NUM_SUBCORES,
    )
    out = pl.kernel(
        _spmm_body,
        out_type=jax.ShapeDtypeStruct(
            (NUM_CORES, NUM_SUBCORES, ROWS_PER_TILE, D), jnp.float32),
        mesh=mesh,
        compiler_params=pltpu.CompilerParams(use_tc_tiling_on_sc=False),
        scratch_types=[
            pltpu.VMEM((3, CHUNK), jnp.int32),
            pltpu.VMEM((3, CHUNK), jnp.int32),
            pltpu.VMEM((CHUNK, WPR), jnp.int32),
            pltpu.VMEM((CHUNK, WPR), jnp.int32),
            pltpu.VMEM((CHUNK, D), jnp.float32),
            pltpu.VMEM((CHUNK, D), jnp.float32),
            pltpu.VMEM((2, CHUNK), jnp.int32),
            pltpu.VMEM_SHARED((N_ROWS, D), jnp.float32),
            pltpu.SemaphoreType.DMA,
            pltpu.SemaphoreType.DMA,
            pltpu.SemaphoreType.DMA,
            pltpu.SemaphoreType.DMA,
            pltpu.SemaphoreType.DMA,
            pltpu.SemaphoreType.DMA,
        ],
    )(emb, recs, zeros)

    out = out.reshape(NUM_CORES, N_ROWS, D // 32, 2, 16)
    out = out.transpose(0, 1, 2, 4, 3).reshape(NUM_CORES, N_ROWS, D)
    return (out[0], out[1])
